# Initial kernel scaffold; baseline (speedup 1.0000x reference)
import jax
import jax.numpy as jnp
from jax.experimental import pallas as pl


def kernel(features, edge_weight, W_self, W_neigh, b_sage, bn_gamma, bn_beta, W_fc, b_fc, edge_index):
    raise NotImplementedError("write your pallas kernel here")


# speedup vs baseline: 5.4793x; 5.4793x over previous
import functools

import jax
import jax.numpy as jnp
from jax import lax
from jax.experimental import pallas as pl
from jax.experimental.pallas import tpu as pltpu
from jax.experimental.pallas import tpu_sc as plsc

N = 10000
E = 320000
D_IN = 128
D_PAD = 144
D_OUT = 160

NC = 2
NS = 16
NW = NC * NS
EPW = E // NW
CHUNK = 80
NCHUNK = EPW // CHUNK
ROWS_FULL = 640
ROWS_TAIL = N - 15 * ROWS_FULL


def _edge_body(packed_hbm, w_hbm, feat_hbm, out_hbm,
               packed_v, sidx_v, didx_v, w_v, msgs_v, agg_sh, sem):
  c = lax.axis_index("c")
  s = lax.axis_index("s")
  wid = s * NC + c

  zero16 = jnp.zeros((16,), jnp.float32)

  def zrow(i, carry):
    for d in range(D_PAD // 16):
      msgs_v[i, pl.ds(d * 16, 16)] = zero16
    return carry

  lax.fori_loop(0, CHUNK, zrow, 0)

  @pl.when(s < NS - 1)
  def _():
    def zcopy(k, carry):
      pltpu.sync_copy(msgs_v, agg_sh.at[pl.ds(s * ROWS_FULL + k * CHUNK, CHUNK)])
      return carry
    lax.fori_loop(0, ROWS_FULL // CHUNK, zcopy, 0)

  @pl.when(s == NS - 1)
  def _():
    def zcopy(k, carry):
      pltpu.sync_copy(msgs_v, agg_sh.at[pl.ds(15 * ROWS_FULL + k * CHUNK, CHUNK)])
      return carry
    lax.fori_loop(0, ROWS_TAIL // CHUNK, zcopy, 0)

  pltpu.sync_copy(packed_hbm.at[wid], packed_v)
  pltpu.sync_copy(w_hbm.at[wid], w_v)

  plsc.subcore_barrier()

  def chunk_body(j, carry):
    for k in range(CHUNK // 16):
      p = packed_v[j, pl.ds(k * 16, 16)]
      sidx_v[pl.ds(k * 16, 16)] = lax.shift_right_logical(p, 14)
      didx_v[pl.ds(k * 16, 16)] = lax.bitwise_and(p, jnp.int32(0x3FFF))

    pltpu.async_copy(feat_hbm.at[sidx_v], msgs_v, sem).wait()

    def edge(e, ecarry):
      wv = plsc.load_gather(w_v, [jnp.full((16,), j * CHUNK + e, jnp.int32)])
      for d in range(D_IN // 16):
        msgs_v[e, pl.ds(d * 16, 16)] = msgs_v[e, pl.ds(d * 16, 16)] * wv
      return ecarry

    lax.fori_loop(0, CHUNK, edge, 0)

    pltpu.sync_copy(msgs_v, agg_sh.at[didx_v], add=True)
    return carry

  lax.fori_loop(0, NCHUNK, chunk_body, 0)

  plsc.subcore_barrier()

  @pl.when(s < NS - 1)
  def _():
    pltpu.sync_copy(agg_sh.at[pl.ds(s * ROWS_FULL, ROWS_FULL)],
                    out_hbm.at[c, pl.ds(s * ROWS_FULL, ROWS_FULL)])

  @pl.when(s == NS - 1)
  def _():
    pltpu.sync_copy(agg_sh.at[pl.ds(15 * ROWS_FULL, ROWS_TAIL)],
                    out_hbm.at[c, pl.ds(15 * ROWS_FULL, ROWS_TAIL)])


@functools.cache
def _edge_kernel():
  return pl.kernel(
      _edge_body,
      out_type=jax.ShapeDtypeStruct((NC, N, D_PAD), jnp.float32),
      mesh=plsc.VectorSubcoreMesh(core_axis_name="c", subcore_axis_name="s",
                                  num_cores=NC, num_subcores=NS),
      compiler_params=pltpu.CompilerParams(needs_layout_passes=False,
                                           use_tc_tiling_on_sc=False),
      scratch_types=[
          pltpu.VMEM((NCHUNK, CHUNK), jnp.int32),
          pltpu.VMEM((CHUNK,), jnp.int32),
          pltpu.VMEM((CHUNK,), jnp.int32),
          pltpu.VMEM((EPW,), jnp.float32),
          pltpu.VMEM((CHUNK, D_PAD), jnp.float32),
          pltpu.VMEM_SHARED((N, D_PAD), jnp.float32),
          pltpu.SemaphoreType.DMA,
      ],
  )


def _dense_body(feat_ref, h2_ref, deg2_ref, wself_ref, wneigh_ref, bsage_ref,
                gamma_ref, beta_ref, wfc_ref, bfc_ref, out_ref):
  h = h2_ref[0] + h2_ref[1]
  deg = deg2_ref[0] + deg2_ref[1]
  inv = 1.0 / jnp.maximum(deg, 1.0)
  h_neigh = jnp.dot(h * inv, wneigh_ref[...].T,
                    preferred_element_type=jnp.float32)
  rst = jnp.dot(feat_ref[...], wself_ref[...].T,
                preferred_element_type=jnp.float32)
  rst = rst + h_neigh + bsage_ref[...][None, :]
  rst = jnp.maximum(rst, 0.0)
  mean = jnp.mean(rst, axis=0, keepdims=True)
  var = jnp.mean((rst - mean) * (rst - mean), axis=0, keepdims=True)
  rst = (rst - mean) * jax.lax.rsqrt(var + 1e-5)
  rst = rst * gamma_ref[...][None, :] + beta_ref[...][None, :]
  out_ref[...] = jnp.dot(rst, wfc_ref[...].T,
                         preferred_element_type=jnp.float32) + bfc_ref[...][None, :]


def kernel(features, edge_weight, W_self, W_neigh, b_sage, bn_gamma, bn_beta,
           W_fc, b_fc, edge_index):
  feat_ext = jnp.concatenate(
      [features,
       jnp.ones((N, 1), jnp.float32),
       jnp.zeros((N, D_PAD - D_IN - 1), jnp.float32)], axis=1)

  packed = ((edge_index[0] << 14) | edge_index[1]).reshape(NW, NCHUNK, CHUNK)
  w3 = edge_weight.reshape(NW, EPW)

  agg2 = _edge_kernel()(packed, w3, feat_ext)

  h2 = agg2[:, :, :D_IN]
  deg2 = agg2[:, :, D_IN:D_IN + 1]

  out = pl.pallas_call(
      _dense_body,
      out_shape=jax.ShapeDtypeStruct((N, D_OUT), jnp.float32),
  )(features, h2, deg2, W_self, W_neigh, b_sage, bn_gamma, bn_beta, W_fc, b_fc)
  return out


# speedup vs baseline: 9.7265x; 1.7751x over previous
import functools

import jax
import jax.numpy as jnp
from jax import lax
from jax.experimental import pallas as pl
from jax.experimental.pallas import tpu as pltpu
from jax.experimental.pallas import tpu_sc as plsc

N = 10000
E = 320000
D_IN = 128
D_PAD = 144
D_OUT = 160

NC = 2
NS = 16
NW = NC * NS
EPW = E // NW
CHUNK = 80
NCHUNK = EPW // CHUNK
ROWS_FULL = 640
ROWS_TAIL = N - 15 * ROWS_FULL


def _edge_body(packed_hbm, w_hbm, feat_hbm, out_hbm,
               pbuf_v, sidx_v, didx_v, w_v, msgs_v, agg_sh, gsems, isems):
  c = lax.axis_index("c")
  s = lax.axis_index("s")
  wid = s * NC + c

  zero16 = jnp.zeros((16,), jnp.float32)

  def zrow(i, carry):
    for d in range(D_PAD // 16):
      msgs_v[0, i, pl.ds(d * 16, 16)] = zero16
    return carry

  lax.fori_loop(0, CHUNK, zrow, 0)

  @pl.when(s < NS - 1)
  def _():
    def zcopy(k, carry):
      pltpu.sync_copy(msgs_v.at[0],
                      agg_sh.at[pl.ds(s * ROWS_FULL + k * CHUNK, CHUNK)])
      return carry
    lax.fori_loop(0, ROWS_FULL // CHUNK, zcopy, 0)

  @pl.when(s == NS - 1)
  def _():
    def zcopy(k, carry):
      pltpu.sync_copy(msgs_v.at[0],
                      agg_sh.at[pl.ds(15 * ROWS_FULL + k * CHUNK, CHUNK)])
      return carry
    lax.fori_loop(0, ROWS_TAIL // CHUNK, zcopy, 0)

  plsc.subcore_barrier()

  def unpack(b):
    for k in range(CHUNK // 16):
      p = pbuf_v[b, pl.ds(k * 16, 16)]
      sidx_v[b, pl.ds(k * 16, 16)] = lax.shift_right_logical(p, 14)
      didx_v[b, pl.ds(k * 16, 16)] = lax.bitwise_and(p, jnp.int32(0x3FFF))

  def gather(b):
    return pltpu.make_async_copy(feat_hbm.at[sidx_v.at[b]], msgs_v.at[b],
                                 gsems.at[b])

  def idx_copy(j, b):
    return pltpu.make_async_copy(packed_hbm.at[wid, j], pbuf_v.at[b],
                                 isems.at[b])

  def w_copy(j, b):
    return pltpu.make_async_copy(w_hbm.at[wid, j],
                                 w_v.at[pl.ds(b * CHUNK, CHUNK)],
                                 isems.at[b])

  pltpu.sync_copy(packed_hbm.at[wid, 0], pbuf_v.at[0])
  pltpu.sync_copy(w_hbm.at[wid, 0], w_v.at[pl.ds(0, CHUNK)])
  unpack(0)
  gather(0).start()
  idx_copy(1, 1).start()
  w_copy(1, 1).start()

  def chunk_body(j, carry):
    b = lax.rem(j, 2)
    nb = 1 - b

    @pl.when(j + 1 < NCHUNK)
    def _():
      idx_copy(j + 1, nb).wait()
      w_copy(j + 1, nb).wait()
      unpack(nb)
      gather(nb).start()

    gather(b).wait()

    @plsc.parallel_loop(0, CHUNK, unroll=4)
    def _(e):
      wv = plsc.load_gather(w_v, [jnp.full((16,), b * CHUNK + e, jnp.int32)])
      for d in range(D_IN // 16):
        msgs_v[b, e, pl.ds(d * 16, 16)] = msgs_v[b, e, pl.ds(d * 16, 16)] * wv

    @pl.when(j + 2 < NCHUNK)
    def _():
      idx_copy(j + 2, b).start()
      w_copy(j + 2, b).start()

    pltpu.sync_copy(msgs_v.at[b], agg_sh.at[didx_v.at[b]], add=True)
    return carry

  lax.fori_loop(0, NCHUNK, chunk_body, 0)

  plsc.subcore_barrier()

  @pl.when(s < NS - 1)
  def _():
    pltpu.sync_copy(agg_sh.at[pl.ds(s * ROWS_FULL, ROWS_FULL)],
                    out_hbm.at[c, pl.ds(s * ROWS_FULL, ROWS_FULL)])

  @pl.when(s == NS - 1)
  def _():
    pltpu.sync_copy(agg_sh.at[pl.ds(15 * ROWS_FULL, ROWS_TAIL)],
                    out_hbm.at[c, pl.ds(15 * ROWS_FULL, ROWS_TAIL)])


@functools.cache
def _edge_kernel():
  return pl.kernel(
      _edge_body,
      out_type=jax.ShapeDtypeStruct((NC, N, D_PAD), jnp.float32),
      mesh=plsc.VectorSubcoreMesh(core_axis_name="c", subcore_axis_name="s",
                                  num_cores=NC, num_subcores=NS),
      compiler_params=pltpu.CompilerParams(needs_layout_passes=False,
                                           use_tc_tiling_on_sc=False),
      scratch_types=[
          pltpu.VMEM((2, CHUNK), jnp.int32),
          pltpu.VMEM((2, CHUNK), jnp.int32),
          pltpu.VMEM((2, CHUNK), jnp.int32),
          pltpu.VMEM((2 * CHUNK,), jnp.float32),
          pltpu.VMEM((2, CHUNK, D_PAD), jnp.float32),
          pltpu.VMEM_SHARED((N, D_PAD), jnp.float32),
          pltpu.SemaphoreType.DMA((2,)),
          pltpu.SemaphoreType.DMA((2,)),
      ],
  )


def _dense_body(feat_ref, agg2_ref, wself_ref, wneigh_ref, bsage_ref,
                gamma_ref, beta_ref, wfc_ref, bfc_ref, out_ref):
  agg = agg2_ref[0] + agg2_ref[1]
  h = agg[:, :D_IN]
  deg = agg[:, D_IN:D_IN + 1]
  inv = 1.0 / jnp.maximum(deg, 1.0)
  h_neigh = jnp.dot(h * inv, wneigh_ref[...].T,
                    preferred_element_type=jnp.float32)
  rst = jnp.dot(feat_ref[...], wself_ref[...].T,
                preferred_element_type=jnp.float32)
  rst = rst + h_neigh + bsage_ref[...][None, :]
  rst = jnp.maximum(rst, 0.0)
  mean = jnp.mean(rst, axis=0, keepdims=True)
  var = jnp.mean((rst - mean) * (rst - mean), axis=0, keepdims=True)
  rst = (rst - mean) * jax.lax.rsqrt(var + 1e-5)
  rst = rst * gamma_ref[...][None, :] + beta_ref[...][None, :]
  out_ref[...] = jnp.dot(rst, wfc_ref[...].T,
                         preferred_element_type=jnp.float32) + bfc_ref[...][None, :]


def kernel(features, edge_weight, W_self, W_neigh, b_sage, bn_gamma, bn_beta,
           W_fc, b_fc, edge_index):
  feat_ext = jnp.concatenate(
      [features,
       jnp.ones((N, 1), jnp.float32),
       jnp.zeros((N, D_PAD - D_IN - 1), jnp.float32)], axis=1)

  packed = ((edge_index[0] << 14) | edge_index[1]).reshape(NW, NCHUNK, CHUNK)
  w3 = edge_weight.reshape(NW, NCHUNK, CHUNK)

  agg2 = _edge_kernel()(packed, w3, feat_ext)

  out = pl.pallas_call(
      _dense_body,
      out_shape=jax.ShapeDtypeStruct((N, D_OUT), jnp.float32),
  )(features, agg2, W_self, W_neigh, b_sage, bn_gamma, bn_beta, W_fc, b_fc)
  return out


# speedup vs baseline: 11.1967x; 1.1512x over previous
import functools

import jax
import jax.numpy as jnp
from jax import lax
from jax.experimental import pallas as pl
from jax.experimental.pallas import tpu as pltpu
from jax.experimental.pallas import tpu_sc as plsc

N = 10000
E = 320000
D_IN = 128
D_DEG = 16
D_OUT = 160

NC = 2
NS = 16
NW = NC * NS
EPW = E // NW
CHUNK = 80
NCHUNK = EPW // CHUNK
ROWS_FULL = 640
ROWS_TAIL = N - 15 * ROWS_FULL


def _edge_body(src_hbm, dst_hbm, w_hbm, feat_hbm, outh_hbm, outd_hbm,
               sidx_v, didx_v, w_v, msgs_v, ones_v, zdeg_v,
               aggh_sh, aggd_sh, gsems, isems, hsems, dsems):
  c = lax.axis_index("c")
  s = lax.axis_index("s")
  wid = s * NC + c

  zero16 = jnp.zeros((16,), jnp.float32)
  one16 = jnp.ones((16,), jnp.float32)

  def initrow(i, carry):
    for d in range(D_IN // 16):
      msgs_v[0, i, pl.ds(d * 16, 16)] = zero16
    ones_v[i, pl.ds(0, 16)] = one16
    zdeg_v[i, pl.ds(0, 16)] = zero16
    return carry

  lax.fori_loop(0, CHUNK, initrow, 0)

  @pl.when(s < NS - 1)
  def _():
    def zcopy(k, carry):
      pltpu.sync_copy(msgs_v.at[0],
                      aggh_sh.at[pl.ds(s * ROWS_FULL + k * CHUNK, CHUNK)])
      pltpu.sync_copy(zdeg_v,
                      aggd_sh.at[pl.ds(s * ROWS_FULL + k * CHUNK, CHUNK)])
      return carry
    lax.fori_loop(0, ROWS_FULL // CHUNK, zcopy, 0)

  @pl.when(s == NS - 1)
  def _():
    def zcopy(k, carry):
      pltpu.sync_copy(msgs_v.at[0],
                      aggh_sh.at[pl.ds(15 * ROWS_FULL + k * CHUNK, CHUNK)])
      pltpu.sync_copy(zdeg_v,
                      aggd_sh.at[pl.ds(15 * ROWS_FULL + k * CHUNK, CHUNK)])
      return carry
    lax.fori_loop(0, ROWS_TAIL // CHUNK, zcopy, 0)

  plsc.subcore_barrier()

  def gather(j3):
    return pltpu.make_async_copy(feat_hbm.at[sidx_v.at[j3]], msgs_v.at[j3],
                                 gsems.at[j3])

  def src_copy(j, j3, b2):
    return pltpu.make_async_copy(src_hbm.at[wid, j], sidx_v.at[j3],
                                 isems.at[b2])

  def dst_copy(j, j3, b2):
    return pltpu.make_async_copy(dst_hbm.at[wid, j], didx_v.at[j3],
                                 isems.at[b2])

  def w_copy(j, b2):
    return pltpu.make_async_copy(w_hbm.at[wid, j],
                                 w_v.at[pl.ds(b2 * CHUNK, CHUNK)],
                                 isems.at[b2])

  def scat_h(j3):
    return pltpu.make_async_copy(msgs_v.at[j3], aggh_sh.at[didx_v.at[j3]],
                                 hsems.at[j3])

  def scat_d(j3):
    return pltpu.make_async_copy(ones_v, aggd_sh.at[didx_v.at[j3]],
                                 dsems.at[j3])

  pltpu.sync_copy(src_hbm.at[wid, 0], sidx_v.at[0])
  pltpu.sync_copy(dst_hbm.at[wid, 0], didx_v.at[0])
  pltpu.sync_copy(w_hbm.at[wid, 0], w_v.at[pl.ds(0, CHUNK)])
  gather(0).start()
  src_copy(1, 1, 1).start()
  dst_copy(1, 1, 1).start()
  w_copy(1, 1).start()

  def chunk_body(j, carry):
    r = lax.rem(j, 3)
    b = lax.rem(j, 2)
    nr = lax.rem(j + 1, 3)
    nb = 1 - b

    @pl.when(j + 1 < NCHUNK)
    def _():
      src_copy(j + 1, nr, nb).wait()
      dst_copy(j + 1, nr, nb).wait()
      w_copy(j + 1, nb).wait()
      gather(nr).start()

    gather(r).wait()

    @plsc.parallel_loop(0, CHUNK, unroll=4)
    def _(e):
      wv = plsc.load_gather(w_v, [jnp.full((16,), b * CHUNK + e, jnp.int32)])
      for d in range(D_IN // 16):
        msgs_v[r, e, pl.ds(d * 16, 16)] = msgs_v[r, e, pl.ds(d * 16, 16)] * wv

    @pl.when(j >= 1)
    def _():
      scat_h(lax.rem(j + 2, 3)).wait()
      scat_d(lax.rem(j + 2, 3)).wait()

    @pl.when(j + 2 < NCHUNK)
    def _():
      src_copy(j + 2, lax.rem(j + 2, 3), b).start()
      dst_copy(j + 2, lax.rem(j + 2, 3), b).start()
      w_copy(j + 2, b).start()

    scat_h(r).start(add=True)
    scat_d(r).start(add=True)
    return carry

  lax.fori_loop(0, NCHUNK, chunk_body, 0)

  scat_h((NCHUNK - 1) % 3).wait()
  scat_d((NCHUNK - 1) % 3).wait()

  plsc.subcore_barrier()

  @pl.when(s < NS - 1)
  def _():
    pltpu.sync_copy(aggh_sh.at[pl.ds(s * ROWS_FULL, ROWS_FULL)],
                    outh_hbm.at[c, pl.ds(s * ROWS_FULL, ROWS_FULL)])
    pltpu.sync_copy(aggd_sh.at[pl.ds(s * ROWS_FULL, ROWS_FULL)],
                    outd_hbm.at[c, pl.ds(s * ROWS_FULL, ROWS_FULL)])

  @pl.when(s == NS - 1)
  def _():
    pltpu.sync_copy(aggh_sh.at[pl.ds(15 * ROWS_FULL, ROWS_TAIL)],
                    outh_hbm.at[c, pl.ds(15 * ROWS_FULL, ROWS_TAIL)])
    pltpu.sync_copy(aggd_sh.at[pl.ds(15 * ROWS_FULL, ROWS_TAIL)],
                    outd_hbm.at[c, pl.ds(15 * ROWS_FULL, ROWS_TAIL)])


@functools.cache
def _edge_kernel():
  return pl.kernel(
      _edge_body,
      out_type=[
          jax.ShapeDtypeStruct((NC, N, D_IN), jnp.float32),
          jax.ShapeDtypeStruct((NC, N, D_DEG), jnp.float32),
      ],
      mesh=plsc.VectorSubcoreMesh(core_axis_name="c", subcore_axis_name="s",
                                  num_cores=NC, num_subcores=NS),
      compiler_params=pltpu.CompilerParams(needs_layout_passes=False,
                                           use_tc_tiling_on_sc=False),
      scratch_types=[
          pltpu.VMEM((3, CHUNK), jnp.int32),
          pltpu.VMEM((3, CHUNK), jnp.int32),
          pltpu.VMEM((2 * CHUNK,), jnp.float32),
          pltpu.VMEM((3, CHUNK, D_IN), jnp.float32),
          pltpu.VMEM((CHUNK, D_DEG), jnp.float32),
          pltpu.VMEM((CHUNK, D_DEG), jnp.float32),
          pltpu.VMEM_SHARED((N, D_IN), jnp.float32),
          pltpu.VMEM_SHARED((N, D_DEG), jnp.float32),
          pltpu.SemaphoreType.DMA((3,)),
          pltpu.SemaphoreType.DMA((2,)),
          pltpu.SemaphoreType.DMA((3,)),
          pltpu.SemaphoreType.DMA((3,)),
      ],
  )


def _dense_body(feat_ref, h2_ref, deg2_ref, wself_ref, wneigh_ref, bsage_ref,
                gamma_ref, beta_ref, wfc_ref, bfc_ref, out_ref):
  h = h2_ref[0] + h2_ref[1]
  deg = deg2_ref[0, :, 0:1] + deg2_ref[1, :, 0:1]
  inv = 1.0 / jnp.maximum(deg, 1.0)
  h_neigh = jnp.dot(h * inv, wneigh_ref[...].T,
                    preferred_element_type=jnp.float32)
  rst = jnp.dot(feat_ref[...], wself_ref[...].T,
                preferred_element_type=jnp.float32)
  rst = rst + h_neigh + bsage_ref[...][None, :]
  rst = jnp.maximum(rst, 0.0)
  mean = jnp.mean(rst, axis=0, keepdims=True)
  var = jnp.mean((rst - mean) * (rst - mean), axis=0, keepdims=True)
  rst = (rst - mean) * jax.lax.rsqrt(var + 1e-5)
  rst = rst * gamma_ref[...][None, :] + beta_ref[...][None, :]
  out_ref[...] = jnp.dot(rst, wfc_ref[...].T,
                         preferred_element_type=jnp.float32) + bfc_ref[...][None, :]


def kernel(features, edge_weight, W_self, W_neigh, b_sage, bn_gamma, bn_beta,
           W_fc, b_fc, edge_index):
  src = edge_index[0].reshape(NW, NCHUNK, CHUNK)
  dst = edge_index[1].reshape(NW, NCHUNK, CHUNK)
  w3 = edge_weight.reshape(NW, NCHUNK, CHUNK)

  h2, deg2 = _edge_kernel()(src, dst, w3, features)

  out = pl.pallas_call(
      _dense_body,
      out_shape=jax.ShapeDtypeStruct((N, D_OUT), jnp.float32),
  )(features, h2, deg2, W_self, W_neigh, b_sage, bn_gamma, bn_beta, W_fc, b_fc)
  return out


# speedup vs baseline: 12.0043x; 1.0721x over previous
import functools

import jax
import jax.numpy as jnp
from jax import lax
from jax.experimental import pallas as pl
from jax.experimental.pallas import tpu as pltpu
from jax.experimental.pallas import tpu_sc as plsc

N = 10000
E = 320000
D_IN = 128
D_DEG = 16
D_OUT = 160

NC = 2
NS = 16
NW = NC * NS
EPW = E // NW
CHUNK = 80
NCHUNK = EPW // CHUNK
ROWS_FULL = 640
ROWS_TAIL = N - 15 * ROWS_FULL


def _edge_body(src_hbm, dst_hbm, w_hbm, feat_hbm, outh_hbm, outd_hbm,
               sidx_v, didx_v, w_v, msgs16_v, msgsf_v, ones_v, zdeg_v,
               aggh_sh, aggd_sh, gsems, isems, hsems, dsems):
  c = lax.axis_index("c")
  s = lax.axis_index("s")
  wid = s * NC + c

  zero16 = jnp.zeros((16,), jnp.float32)
  one16 = jnp.ones((16,), jnp.float32)

  def initrow(i, carry):
    for d in range(D_IN // 16):
      msgsf_v[0, i, pl.ds(d * 16, 16)] = zero16
    ones_v[i, pl.ds(0, 16)] = one16
    zdeg_v[i, pl.ds(0, 16)] = zero16
    return carry

  lax.fori_loop(0, CHUNK, initrow, 0)

  @pl.when(s < NS - 1)
  def _():
    def zcopy(k, carry):
      pltpu.sync_copy(msgsf_v.at[0],
                      aggh_sh.at[pl.ds(s * ROWS_FULL + k * CHUNK, CHUNK)])
      pltpu.sync_copy(zdeg_v,
                      aggd_sh.at[pl.ds(s * ROWS_FULL + k * CHUNK, CHUNK)])
      return carry
    lax.fori_loop(0, ROWS_FULL // CHUNK, zcopy, 0)

  @pl.when(s == NS - 1)
  def _():
    def zcopy(k, carry):
      pltpu.sync_copy(msgsf_v.at[0],
                      aggh_sh.at[pl.ds(15 * ROWS_FULL + k * CHUNK, CHUNK)])
      pltpu.sync_copy(zdeg_v,
                      aggd_sh.at[pl.ds(15 * ROWS_FULL + k * CHUNK, CHUNK)])
      return carry
    lax.fori_loop(0, ROWS_TAIL // CHUNK, zcopy, 0)

  plsc.subcore_barrier()

  def gather(j3, b2):
    return pltpu.make_async_copy(feat_hbm.at[sidx_v.at[j3]], msgs16_v.at[b2],
                                 gsems.at[b2])

  def src_copy(j, j3, b2):
    return pltpu.make_async_copy(src_hbm.at[wid, j], sidx_v.at[j3],
                                 isems.at[b2])

  def dst_copy(j, j3, b2):
    return pltpu.make_async_copy(dst_hbm.at[wid, j], didx_v.at[j3],
                                 isems.at[b2])

  def w_copy(j, b2):
    return pltpu.make_async_copy(w_hbm.at[wid, j],
                                 w_v.at[pl.ds(b2 * CHUNK, CHUNK)],
                                 isems.at[b2])

  def scat_h(j3, b2):
    return pltpu.make_async_copy(msgsf_v.at[b2], aggh_sh.at[didx_v.at[j3]],
                                 hsems.at[j3])

  def scat_d(j3):
    return pltpu.make_async_copy(ones_v, aggd_sh.at[didx_v.at[j3]],
                                 dsems.at[j3])

  pltpu.sync_copy(src_hbm.at[wid, 0], sidx_v.at[0])
  pltpu.sync_copy(dst_hbm.at[wid, 0], didx_v.at[0])
  pltpu.sync_copy(w_hbm.at[wid, 0], w_v.at[pl.ds(0, CHUNK)])
  gather(0, 0).start()
  src_copy(1, 1, 1).start()
  dst_copy(1, 1, 1).start()
  w_copy(1, 1).start()

  def chunk_body(j, carry):
    r = lax.rem(j, 3)
    b = lax.rem(j, 2)
    nr = lax.rem(j + 1, 3)
    nb = 1 - b

    @pl.when(j + 1 < NCHUNK)
    def _():
      src_copy(j + 1, nr, nb).wait()
      dst_copy(j + 1, nr, nb).wait()
      w_copy(j + 1, nb).wait()
      gather(nr, nb).start()

    gather(r, b).wait()

    @plsc.parallel_loop(0, CHUNK, unroll=2)
    def _(e):
      wv = plsc.load_gather(w_v, [jnp.full((16,), b * CHUNK + e, jnp.int32)])
      for k in range(D_IN // 32):
        x = msgs16_v[b, e, pl.ds(k * 32, 32)]
        lo, hi = plsc.unpack(x, format=plsc.PackFormat.INTERLEAVED)
        msgsf_v[b, e, pl.ds(k * 32, 16)] = lo * wv
        msgsf_v[b, e, pl.ds(k * 32 + 16, 16)] = hi * wv

    @pl.when(j >= 1)
    def _():
      scat_h(lax.rem(j + 2, 3), nb).wait()
      scat_d(lax.rem(j + 2, 3)).wait()

    @pl.when(j + 2 < NCHUNK)
    def _():
      src_copy(j + 2, lax.rem(j + 2, 3), b).start()
      dst_copy(j + 2, lax.rem(j + 2, 3), b).start()
      w_copy(j + 2, b).start()

    scat_h(r, b).start(add=True)
    scat_d(r).start(add=True)
    return carry

  lax.fori_loop(0, NCHUNK, chunk_body, 0)

  scat_h((NCHUNK - 1) % 3, (NCHUNK - 1) % 2).wait()
  scat_d((NCHUNK - 1) % 3).wait()

  plsc.subcore_barrier()

  @pl.when(s < NS - 1)
  def _():
    pltpu.sync_copy(aggh_sh.at[pl.ds(s * ROWS_FULL, ROWS_FULL)],
                    outh_hbm.at[c, pl.ds(s * ROWS_FULL, ROWS_FULL)])
    pltpu.sync_copy(aggd_sh.at[pl.ds(s * ROWS_FULL, ROWS_FULL)],
                    outd_hbm.at[c, pl.ds(s * ROWS_FULL, ROWS_FULL)])

  @pl.when(s == NS - 1)
  def _():
    pltpu.sync_copy(aggh_sh.at[pl.ds(15 * ROWS_FULL, ROWS_TAIL)],
                    outh_hbm.at[c, pl.ds(15 * ROWS_FULL, ROWS_TAIL)])
    pltpu.sync_copy(aggd_sh.at[pl.ds(15 * ROWS_FULL, ROWS_TAIL)],
                    outd_hbm.at[c, pl.ds(15 * ROWS_FULL, ROWS_TAIL)])


@functools.cache
def _edge_kernel():
  return pl.kernel(
      _edge_body,
      out_type=[
          jax.ShapeDtypeStruct((NC, N, D_IN), jnp.float32),
          jax.ShapeDtypeStruct((NC, N, D_DEG), jnp.float32),
      ],
      mesh=plsc.VectorSubcoreMesh(core_axis_name="c", subcore_axis_name="s",
                                  num_cores=NC, num_subcores=NS),
      compiler_params=pltpu.CompilerParams(needs_layout_passes=False,
                                           use_tc_tiling_on_sc=False),
      scratch_types=[
          pltpu.VMEM((3, CHUNK), jnp.int32),
          pltpu.VMEM((3, CHUNK), jnp.int32),
          pltpu.VMEM((2 * CHUNK,), jnp.float32),
          pltpu.VMEM((2, CHUNK, D_IN), jnp.bfloat16),
          pltpu.VMEM((2, CHUNK, D_IN), jnp.float32),
          pltpu.VMEM((CHUNK, D_DEG), jnp.float32),
          pltpu.VMEM((CHUNK, D_DEG), jnp.float32),
          pltpu.VMEM_SHARED((N, D_IN), jnp.float32),
          pltpu.VMEM_SHARED((N, D_DEG), jnp.float32),
          pltpu.SemaphoreType.DMA((2,)),
          pltpu.SemaphoreType.DMA((2,)),
          pltpu.SemaphoreType.DMA((3,)),
          pltpu.SemaphoreType.DMA((3,)),
      ],
  )


def _dense_body(feat_ref, h2_ref, deg2_ref, wself_ref, wneigh_ref, bsage_ref,
                gamma_ref, beta_ref, wfc_ref, bfc_ref, out_ref):
  h = h2_ref[0] + h2_ref[1]
  deg = deg2_ref[0, :, 0:1] + deg2_ref[1, :, 0:1]
  inv = 1.0 / jnp.maximum(deg, 1.0)
  h_neigh = jnp.dot(h * inv, wneigh_ref[...].T,
                    preferred_element_type=jnp.float32)
  rst = jnp.dot(feat_ref[...], wself_ref[...].T,
                preferred_element_type=jnp.float32)
  rst = rst + h_neigh + bsage_ref[...][None, :]
  rst = jnp.maximum(rst, 0.0)
  mean = jnp.mean(rst, axis=0, keepdims=True)
  var = jnp.mean((rst - mean) * (rst - mean), axis=0, keepdims=True)
  rst = (rst - mean) * jax.lax.rsqrt(var + 1e-5)
  rst = rst * gamma_ref[...][None, :] + beta_ref[...][None, :]
  out_ref[...] = jnp.dot(rst, wfc_ref[...].T,
                         preferred_element_type=jnp.float32) + bfc_ref[...][None, :]


_UNPACK_PERM = [32 * (q // 32) + 2 * (q % 32) if q % 32 < 16
                else 32 * (q // 32) + 2 * (q % 32 - 16) + 1
                for q in range(D_IN)]


def kernel(features, edge_weight, W_self, W_neigh, b_sage, bn_gamma, bn_beta,
           W_fc, b_fc, edge_index):
  src = edge_index[0].reshape(NW, NCHUNK, CHUNK)
  dst = edge_index[1].reshape(NW, NCHUNK, CHUNK)
  w3 = edge_weight.reshape(NW, NCHUNK, CHUNK)
  feat16 = features.astype(jnp.bfloat16)

  h2, deg2 = _edge_kernel()(src, dst, w3, feat16)

  wneigh_p = W_neigh[:, jnp.array(_UNPACK_PERM, jnp.int32)]

  out = pl.pallas_call(
      _dense_body,
      out_shape=jax.ShapeDtypeStruct((N, D_OUT), jnp.float32),
  )(features, h2, deg2, W_self, wneigh_p, b_sage, bn_gamma, bn_beta, W_fc, b_fc)
  return out


# speedup vs baseline: 12.2048x; 1.0167x over previous
import functools

import jax
import jax.numpy as jnp
from jax import lax
from jax.experimental import pallas as pl
from jax.experimental.pallas import tpu as pltpu
from jax.experimental.pallas import tpu_sc as plsc

N = 10000
E = 320000
D_IN = 128
D_DEG = 16
D_OUT = 160

NC = 2
NS = 16
NW = NC * NS
EPW = E // NW
CHUNK = 80
NCHUNK = EPW // CHUNK
ROWS_FULL = 640
ROWS_TAIL = N - 15 * ROWS_FULL


def _edge_body(ei_hbm, w_hbm, feat_hbm, outh_hbm, outd_hbm,
               sidx_v, didx_v, w_v, msgs16_v, msgsf_v, ones_v, zdeg_v,
               aggh_sh, aggd_sh, gsems, isems, hsems, dsems):
  c = lax.axis_index("c")
  s = lax.axis_index("s")
  wid = s * NC + c
  base = wid * EPW

  zero16 = jnp.zeros((16,), jnp.float32)
  one16 = jnp.ones((16,), jnp.float32)

  def initrow(i, carry):
    for d in range(D_IN // 16):
      msgsf_v[0, i, pl.ds(d * 16, 16)] = zero16
    ones_v[i, pl.ds(0, 16)] = one16
    zdeg_v[i, pl.ds(0, 16)] = zero16
    return carry

  lax.fori_loop(0, CHUNK, initrow, 0)

  @pl.when(s < NS - 1)
  def _():
    def zcopy(k, carry):
      pltpu.sync_copy(msgsf_v.at[0],
                      aggh_sh.at[pl.ds(s * ROWS_FULL + k * CHUNK, CHUNK)])
      pltpu.sync_copy(zdeg_v,
                      aggd_sh.at[pl.ds(s * ROWS_FULL + k * CHUNK, CHUNK)])
      return carry
    lax.fori_loop(0, ROWS_FULL // CHUNK, zcopy, 0)

  @pl.when(s == NS - 1)
  def _():
    def zcopy(k, carry):
      pltpu.sync_copy(msgsf_v.at[0],
                      aggh_sh.at[pl.ds(15 * ROWS_FULL + k * CHUNK, CHUNK)])
      pltpu.sync_copy(zdeg_v,
                      aggd_sh.at[pl.ds(15 * ROWS_FULL + k * CHUNK, CHUNK)])
      return carry
    lax.fori_loop(0, ROWS_TAIL // CHUNK, zcopy, 0)

  plsc.subcore_barrier()

  def gather(j3, b2):
    return pltpu.make_async_copy(feat_hbm.at[sidx_v.at[j3]], msgs16_v.at[b2],
                                 gsems.at[b2])

  def src_copy(j, j3, b2):
    return pltpu.make_async_copy(ei_hbm.at[0, pl.ds(base + j * CHUNK, CHUNK)],
                                 sidx_v.at[j3], isems.at[b2])

  def dst_copy(j, j3, b2):
    return pltpu.make_async_copy(ei_hbm.at[1, pl.ds(base + j * CHUNK, CHUNK)],
                                 didx_v.at[j3], isems.at[b2])

  def w_copy(j, b2):
    return pltpu.make_async_copy(w_hbm.at[pl.ds(base + j * CHUNK, CHUNK)],
                                 w_v.at[pl.ds(b2 * CHUNK, CHUNK)],
                                 isems.at[b2])

  def scat_h(j3, b2):
    return pltpu.make_async_copy(msgsf_v.at[b2], aggh_sh.at[didx_v.at[j3]],
                                 hsems.at[j3])

  def scat_d(j3):
    return pltpu.make_async_copy(ones_v, aggd_sh.at[didx_v.at[j3]],
                                 dsems.at[j3])

  pltpu.sync_copy(ei_hbm.at[0, pl.ds(base, CHUNK)], sidx_v.at[0])
  pltpu.sync_copy(ei_hbm.at[1, pl.ds(base, CHUNK)], didx_v.at[0])
  pltpu.sync_copy(w_hbm.at[pl.ds(base, CHUNK)], w_v.at[pl.ds(0, CHUNK)])
  gather(0, 0).start()
  src_copy(1, 1, 1).start()
  dst_copy(1, 1, 1).start()
  w_copy(1, 1).start()

  def chunk_body(j, carry):
    r = lax.rem(j, 3)
    b = lax.rem(j, 2)
    nr = lax.rem(j + 1, 3)
    nb = 1 - b

    @pl.when(j + 1 < NCHUNK)
    def _():
      src_copy(j + 1, nr, nb).wait()
      dst_copy(j + 1, nr, nb).wait()
      w_copy(j + 1, nb).wait()
      gather(nr, nb).start()

    gather(r, b).wait()

    @plsc.parallel_loop(0, CHUNK, unroll=2)
    def _(e):
      wv = plsc.load_gather(w_v, [jnp.full((16,), b * CHUNK + e, jnp.int32)])
      for k in range(D_IN // 32):
        x = msgs16_v[b, e, pl.ds(k * 32, 32)]
        lo, hi = plsc.unpack(x, format=plsc.PackFormat.INTERLEAVED)
        msgsf_v[b, e, pl.ds(k * 32, 16)] = lo * wv
        msgsf_v[b, e, pl.ds(k * 32 + 16, 16)] = hi * wv

    @pl.when(j >= 1)
    def _():
      scat_h(lax.rem(j + 2, 3), nb).wait()
      scat_d(lax.rem(j + 2, 3)).wait()

    @pl.when(j + 2 < NCHUNK)
    def _():
      src_copy(j + 2, lax.rem(j + 2, 3), b).start()
      dst_copy(j + 2, lax.rem(j + 2, 3), b).start()
      w_copy(j + 2, b).start()

    scat_h(r, b).start(add=True)
    scat_d(r).start(add=True)
    return carry

  lax.fori_loop(0, NCHUNK, chunk_body, 0)

  scat_h((NCHUNK - 1) % 3, (NCHUNK - 1) % 2).wait()
  scat_d((NCHUNK - 1) % 3).wait()

  plsc.subcore_barrier()

  @pl.when(s < NS - 1)
  def _():
    pltpu.sync_copy(aggh_sh.at[pl.ds(s * ROWS_FULL, ROWS_FULL)],
                    outh_hbm.at[c, pl.ds(s * ROWS_FULL, ROWS_FULL)])
    pltpu.sync_copy(aggd_sh.at[pl.ds(s * ROWS_FULL, ROWS_FULL)],
                    outd_hbm.at[c, pl.ds(s * ROWS_FULL, ROWS_FULL)])

  @pl.when(s == NS - 1)
  def _():
    pltpu.sync_copy(aggh_sh.at[pl.ds(15 * ROWS_FULL, ROWS_TAIL)],
                    outh_hbm.at[c, pl.ds(15 * ROWS_FULL, ROWS_TAIL)])
    pltpu.sync_copy(aggd_sh.at[pl.ds(15 * ROWS_FULL, ROWS_TAIL)],
                    outd_hbm.at[c, pl.ds(15 * ROWS_FULL, ROWS_TAIL)])


@functools.cache
def _edge_kernel():
  return pl.kernel(
      _edge_body,
      out_type=[
          jax.ShapeDtypeStruct((NC, N, D_IN), jnp.float32),
          jax.ShapeDtypeStruct((NC, N, D_DEG), jnp.float32),
      ],
      mesh=plsc.VectorSubcoreMesh(core_axis_name="c", subcore_axis_name="s",
                                  num_cores=NC, num_subcores=NS),
      compiler_params=pltpu.CompilerParams(needs_layout_passes=False,
                                           use_tc_tiling_on_sc=False),
      scratch_types=[
          pltpu.VMEM((3, CHUNK), jnp.int32),
          pltpu.VMEM((3, CHUNK), jnp.int32),
          pltpu.VMEM((2 * CHUNK,), jnp.float32),
          pltpu.VMEM((2, CHUNK, D_IN), jnp.bfloat16),
          pltpu.VMEM((2, CHUNK, D_IN), jnp.float32),
          pltpu.VMEM((CHUNK, D_DEG), jnp.float32),
          pltpu.VMEM((CHUNK, D_DEG), jnp.float32),
          pltpu.VMEM_SHARED((N, D_IN), jnp.float32),
          pltpu.VMEM_SHARED((N, D_DEG), jnp.float32),
          pltpu.SemaphoreType.DMA((2,)),
          pltpu.SemaphoreType.DMA((2,)),
          pltpu.SemaphoreType.DMA((3,)),
          pltpu.SemaphoreType.DMA((3,)),
      ],
  )


def _dense_body(feat_ref, h2_any, deg2_any, wself_ref, wneigh_ref, bsage_ref,
                gamma_ref, beta_ref, wfc_ref, bfc_ref, out_ref,
                h2_ref, deg2_ref, sem):
  pltpu.async_copy(h2_any, h2_ref, sem).wait()
  pltpu.async_copy(deg2_any, deg2_ref, sem).wait()
  h = h2_ref[0] + h2_ref[1]
  deg = deg2_ref[0, :, 0:1] + deg2_ref[1, :, 0:1]
  inv = 1.0 / jnp.maximum(deg, 1.0)
  h_neigh = jnp.dot(h * inv, wneigh_ref[...].T,
                    preferred_element_type=jnp.float32)
  rst = jnp.dot(feat_ref[...], wself_ref[...].T,
                preferred_element_type=jnp.float32)
  rst = rst + h_neigh + bsage_ref[...][None, :]
  rst = jnp.maximum(rst, 0.0)
  mean = jnp.mean(rst, axis=0, keepdims=True)
  var = jnp.mean((rst - mean) * (rst - mean), axis=0, keepdims=True)
  rst = (rst - mean) * jax.lax.rsqrt(var + 1e-5)
  rst = rst * gamma_ref[...][None, :] + beta_ref[...][None, :]
  out_ref[...] = jnp.dot(rst, wfc_ref[...].T,
                         preferred_element_type=jnp.float32) + bfc_ref[...][None, :]


_UNPACK_PERM = [32 * (q // 32) + 2 * (q % 32) if q % 32 < 16
                else 32 * (q // 32) + 2 * (q % 32 - 16) + 1
                for q in range(D_IN)]


def kernel(features, edge_weight, W_self, W_neigh, b_sage, bn_gamma, bn_beta,
           W_fc, b_fc, edge_index):
  feat16 = features.astype(jnp.bfloat16)

  h2, deg2 = _edge_kernel()(edge_index, edge_weight, feat16)

  wneigh_p = W_neigh[:, jnp.array(_UNPACK_PERM, jnp.int32)]

  vspec = pl.BlockSpec(memory_space=pltpu.VMEM)
  out = pl.pallas_call(
      _dense_body,
      out_shape=jax.ShapeDtypeStruct((N, D_OUT), jnp.float32),
      in_specs=[vspec,
                pl.BlockSpec(memory_space=pl.ANY),
                pl.BlockSpec(memory_space=pl.ANY),
                vspec, vspec, vspec, vspec, vspec, vspec, vspec],
      scratch_shapes=[
          pltpu.VMEM((NC, N, D_IN), jnp.float32),
          pltpu.VMEM((NC, N, D_DEG), jnp.float32),
          pltpu.SemaphoreType.DMA,
      ],
  )(features, h2, deg2, W_self, wneigh_p, b_sage, bn_gamma, bn_beta, W_fc, b_fc)
  return out


# speedup vs baseline: 12.2651x; 1.0049x over previous
import functools

import jax
import jax.numpy as jnp
from jax import lax
from jax.experimental import pallas as pl
from jax.experimental.pallas import tpu as pltpu
from jax.experimental.pallas import tpu_sc as plsc

N = 10000
E = 320000
D_IN = 128
D_DEG = 16
D_OUT = 160

NC = 2
NS = 16
NW = NC * NS
EPW = E // NW
CHUNK = 80
NCHUNK = EPW // CHUNK
ROWS_FULL = 640
ROWS_TAIL = N - 15 * ROWS_FULL


def _edge_body(ei_hbm, w_hbm, feat_hbm, outh_hbm, outd_hbm,
               sidx_v, w_v, msgs16_v, msgsf_v, ones_v, zdeg_v,
               aggh_sh, aggd_sh, gsems, isems, hsems, dsems):
  c = lax.axis_index("c")
  s = lax.axis_index("s")
  wid = s * NC + c
  base = wid * EPW

  zero16 = jnp.zeros((16,), jnp.float32)
  one16 = jnp.ones((16,), jnp.float32)

  def initrow(i, carry):
    for d in range(D_IN // 16):
      msgsf_v[0, i, pl.ds(d * 16, 16)] = zero16
    ones_v[i, pl.ds(0, 16)] = one16
    zdeg_v[i, pl.ds(0, 16)] = zero16
    return carry

  lax.fori_loop(0, CHUNK, initrow, 0)

  @pl.when(s < NS - 1)
  def _():
    def zcopy(k, carry):
      pltpu.sync_copy(msgsf_v.at[0],
                      aggh_sh.at[pl.ds(s * ROWS_FULL + k * CHUNK, CHUNK)])
      pltpu.sync_copy(zdeg_v,
                      aggd_sh.at[pl.ds(s * ROWS_FULL + k * CHUNK, CHUNK)])
      return carry
    lax.fori_loop(0, ROWS_FULL // CHUNK, zcopy, 0)

  @pl.when(s == NS - 1)
  def _():
    def zcopy(k, carry):
      pltpu.sync_copy(msgsf_v.at[0],
                      aggh_sh.at[pl.ds(15 * ROWS_FULL + k * CHUNK, CHUNK)])
      pltpu.sync_copy(zdeg_v,
                      aggd_sh.at[pl.ds(15 * ROWS_FULL + k * CHUNK, CHUNK)])
      return carry
    lax.fori_loop(0, ROWS_TAIL // CHUNK, zcopy, 0)

  plsc.subcore_barrier()

  def gather(j3, b2):
    return pltpu.make_async_copy(feat_hbm.at[sidx_v.at[j3, 0]],
                                 msgs16_v.at[b2], gsems.at[b2])

  def idx_copy(j, j3, b2):
    return pltpu.make_async_copy(ei_hbm.at[:, pl.ds(base + j * CHUNK, CHUNK)],
                                 sidx_v.at[j3], isems.at[b2])

  def w_copy(j, b2):
    return pltpu.make_async_copy(w_hbm.at[pl.ds(base + j * CHUNK, CHUNK)],
                                 w_v.at[pl.ds(b2 * CHUNK, CHUNK)],
                                 isems.at[b2])

  def scat_h(j3, b2):
    return pltpu.make_async_copy(msgsf_v.at[b2], aggh_sh.at[sidx_v.at[j3, 1]],
                                 hsems.at[j3])

  def scat_d(j3):
    return pltpu.make_async_copy(ones_v, aggd_sh.at[sidx_v.at[j3, 1]],
                                 dsems.at[j3])

  pltpu.sync_copy(ei_hbm.at[:, pl.ds(base, CHUNK)], sidx_v.at[0])
  pltpu.sync_copy(w_hbm.at[pl.ds(base, CHUNK)], w_v.at[pl.ds(0, CHUNK)])
  gather(0, 0).start()
  idx_copy(1, 1, 1).start()
  w_copy(1, 1).start()

  def chunk_body(j, carry):
    r = lax.rem(j, 3)
    b = lax.rem(j, 2)
    nr = lax.rem(j + 1, 3)
    nb = 1 - b

    @pl.when(j + 1 < NCHUNK)
    def _():
      idx_copy(j + 1, nr, nb).wait()
      w_copy(j + 1, nb).wait()
      gather(nr, nb).start()

    gather(r, b).wait()

    @plsc.parallel_loop(0, CHUNK, unroll=4)
    def _(e):
      wv = plsc.load_gather(w_v, [jnp.full((16,), b * CHUNK + e, jnp.int32)])
      for k in range(D_IN // 32):
        x = msgs16_v[b, e, pl.ds(k * 32, 32)]
        lo, hi = plsc.unpack(x, format=plsc.PackFormat.INTERLEAVED)
        msgsf_v[b, e, pl.ds(k * 32, 16)] = lo * wv
        msgsf_v[b, e, pl.ds(k * 32 + 16, 16)] = hi * wv

    @pl.when(j >= 1)
    def _():
      scat_h(lax.rem(j + 2, 3), nb).wait()
      scat_d(lax.rem(j + 2, 3)).wait()

    @pl.when(j + 2 < NCHUNK)
    def _():
      idx_copy(j + 2, lax.rem(j + 2, 3), b).start()
      w_copy(j + 2, b).start()

    scat_h(r, b).start(add=True)
    scat_d(r).start(add=True)
    return carry

  lax.fori_loop(0, NCHUNK, chunk_body, 0)

  scat_h((NCHUNK - 1) % 3, (NCHUNK - 1) % 2).wait()
  scat_d((NCHUNK - 1) % 3).wait()

  plsc.subcore_barrier()

  @pl.when(s < NS - 1)
  def _():
    pltpu.sync_copy(aggh_sh.at[pl.ds(s * ROWS_FULL, ROWS_FULL)],
                    outh_hbm.at[c, pl.ds(s * ROWS_FULL, ROWS_FULL)])
    pltpu.sync_copy(aggd_sh.at[pl.ds(s * ROWS_FULL, ROWS_FULL)],
                    outd_hbm.at[c, pl.ds(s * ROWS_FULL, ROWS_FULL)])

  @pl.when(s == NS - 1)
  def _():
    pltpu.sync_copy(aggh_sh.at[pl.ds(15 * ROWS_FULL, ROWS_TAIL)],
                    outh_hbm.at[c, pl.ds(15 * ROWS_FULL, ROWS_TAIL)])
    pltpu.sync_copy(aggd_sh.at[pl.ds(15 * ROWS_FULL, ROWS_TAIL)],
                    outd_hbm.at[c, pl.ds(15 * ROWS_FULL, ROWS_TAIL)])


@functools.cache
def _edge_kernel():
  return pl.kernel(
      _edge_body,
      out_type=[
          jax.ShapeDtypeStruct((NC, N, D_IN), jnp.float32),
          jax.ShapeDtypeStruct((NC, N, D_DEG), jnp.float32),
      ],
      mesh=plsc.VectorSubcoreMesh(core_axis_name="c", subcore_axis_name="s",
                                  num_cores=NC, num_subcores=NS),
      compiler_params=pltpu.CompilerParams(needs_layout_passes=False,
                                           use_tc_tiling_on_sc=False),
      scratch_types=[
          pltpu.VMEM((3, 2, CHUNK), jnp.int32),
          pltpu.VMEM((2 * CHUNK,), jnp.float32),
          pltpu.VMEM((2, CHUNK, D_IN), jnp.bfloat16),
          pltpu.VMEM((2, CHUNK, D_IN), jnp.float32),
          pltpu.VMEM((CHUNK, D_DEG), jnp.float32),
          pltpu.VMEM((CHUNK, D_DEG), jnp.float32),
          pltpu.VMEM_SHARED((N, D_IN), jnp.float32),
          pltpu.VMEM_SHARED((N, D_DEG), jnp.float32),
          pltpu.SemaphoreType.DMA((2,)),
          pltpu.SemaphoreType.DMA((2,)),
          pltpu.SemaphoreType.DMA((3,)),
          pltpu.SemaphoreType.DMA((3,)),
      ],
  )


def _dense_body(feat_ref, h2_any, deg2_any, wself_ref, wneigh_ref, bsage_ref,
                gamma_ref, beta_ref, wfc_ref, bfc_ref, out_ref,
                h2_ref, deg2_ref, sem):
  pltpu.async_copy(h2_any, h2_ref, sem).wait()
  pltpu.async_copy(deg2_any, deg2_ref, sem).wait()
  h = h2_ref[0] + h2_ref[1]
  deg = deg2_ref[0, :, 0:1] + deg2_ref[1, :, 0:1]
  inv = 1.0 / jnp.maximum(deg, 1.0)
  h_neigh = jnp.dot(h * inv, wneigh_ref[...].T,
                    preferred_element_type=jnp.float32)
  rst = jnp.dot(feat_ref[...], wself_ref[...].T,
                preferred_element_type=jnp.float32)
  rst = rst + h_neigh + bsage_ref[...][None, :]
  rst = jnp.maximum(rst, 0.0)
  mean = jnp.mean(rst, axis=0, keepdims=True)
  var = jnp.mean((rst - mean) * (rst - mean), axis=0, keepdims=True)
  rst = (rst - mean) * jax.lax.rsqrt(var + 1e-5)
  rst = rst * gamma_ref[...][None, :] + beta_ref[...][None, :]
  out_ref[...] = jnp.dot(rst, wfc_ref[...].T,
                         preferred_element_type=jnp.float32) + bfc_ref[...][None, :]


_UNPACK_PERM = [32 * (q // 32) + 2 * (q % 32) if q % 32 < 16
                else 32 * (q // 32) + 2 * (q % 32 - 16) + 1
                for q in range(D_IN)]


def kernel(features, edge_weight, W_self, W_neigh, b_sage, bn_gamma, bn_beta,
           W_fc, b_fc, edge_index):
  feat16 = features.astype(jnp.bfloat16)

  h2, deg2 = _edge_kernel()(edge_index, edge_weight, feat16)

  wneigh_p = W_neigh[:, jnp.array(_UNPACK_PERM, jnp.int32)]

  vspec = pl.BlockSpec(memory_space=pltpu.VMEM)
  out = pl.pallas_call(
      _dense_body,
      out_shape=jax.ShapeDtypeStruct((N, D_OUT), jnp.float32),
      in_specs=[vspec,
                pl.BlockSpec(memory_space=pl.ANY),
                pl.BlockSpec(memory_space=pl.ANY),
                vspec, vspec, vspec, vspec, vspec, vspec, vspec],
      scratch_shapes=[
          pltpu.VMEM((NC, N, D_IN), jnp.float32),
          pltpu.VMEM((NC, N, D_DEG), jnp.float32),
          pltpu.SemaphoreType.DMA,
      ],
  )(features, h2, deg2, W_self, wneigh_p, b_sage, bn_gamma, bn_beta, W_fc, b_fc)
  return out


# speedup vs baseline: 12.6611x; 1.0323x over previous
import functools

import jax
import jax.numpy as jnp
from jax import lax
from jax.experimental import pallas as pl
from jax.experimental.pallas import tpu as pltpu
from jax.experimental.pallas import tpu_sc as plsc

N = 10000
E = 320000
D_IN = 128
D_DEG = 16
D_OUT = 160

NC = 2
NS = 16
NW = NC * NS
EPW = E // NW
CHUNK = 80
NCHUNK = EPW // CHUNK
ROWS_FULL = 640
ROWS_TAIL = N - 15 * ROWS_FULL


def _edge_body(ei_hbm, w_hbm, feat_hbm, outh_hbm, outd_hbm,
               sidx_v, w_v, msgs16_v, msgsf_v, ones_v, zdeg_v,
               aggh_sh, aggd_sh, gsems, isems, hsems, dsems):
  c = lax.axis_index("c")
  s = lax.axis_index("s")
  wid = s * NC + c
  base = wid * EPW

  zero16 = jnp.zeros((16,), jnp.float32)
  one16 = jnp.ones((16,), jnp.float32)

  def initrow(i, carry):
    for d in range(D_IN // 16):
      msgsf_v[0, i, pl.ds(d * 16, 16)] = zero16
    ones_v[i, pl.ds(0, 16)] = one16
    zdeg_v[i, pl.ds(0, 16)] = zero16
    return carry

  lax.fori_loop(0, CHUNK, initrow, 0)

  @pl.when(s < NS - 1)
  def _():
    def zcopy(k, carry):
      pltpu.sync_copy(msgsf_v.at[0],
                      aggh_sh.at[pl.ds(s * ROWS_FULL + k * CHUNK, CHUNK)])
      pltpu.sync_copy(zdeg_v,
                      aggd_sh.at[pl.ds(s * ROWS_FULL + k * CHUNK, CHUNK)])
      return carry
    lax.fori_loop(0, ROWS_FULL // CHUNK, zcopy, 0)

  @pl.when(s == NS - 1)
  def _():
    def zcopy(k, carry):
      pltpu.sync_copy(msgsf_v.at[0],
                      aggh_sh.at[pl.ds(15 * ROWS_FULL + k * CHUNK, CHUNK)])
      pltpu.sync_copy(zdeg_v,
                      aggd_sh.at[pl.ds(15 * ROWS_FULL + k * CHUNK, CHUNK)])
      return carry
    lax.fori_loop(0, ROWS_TAIL // CHUNK, zcopy, 0)

  plsc.subcore_barrier()

  def gather(j):
    return pltpu.make_async_copy(feat_hbm.at[sidx_v.at[lax.rem(j, 4), 0]],
                                 msgs16_v.at[lax.rem(j, 3)],
                                 gsems.at[lax.rem(j, 3)])

  def idx_copy(j):
    return pltpu.make_async_copy(ei_hbm.at[:, pl.ds(base + j * CHUNK, CHUNK)],
                                 sidx_v.at[lax.rem(j, 4)],
                                 isems.at[lax.rem(j, 2)])

  def w_copy(j):
    return pltpu.make_async_copy(w_hbm.at[pl.ds(base + j * CHUNK, CHUNK)],
                                 w_v.at[pl.ds(lax.rem(j, 3) * CHUNK, CHUNK)],
                                 isems.at[lax.rem(j, 2)])

  def scat_h(j):
    return pltpu.make_async_copy(msgsf_v.at[lax.rem(j, 2)],
                                 aggh_sh.at[sidx_v.at[lax.rem(j, 4), 1]],
                                 hsems.at[lax.rem(j, 3)])

  def scat_d(j):
    return pltpu.make_async_copy(ones_v, aggd_sh.at[sidx_v.at[lax.rem(j, 4), 1]],
                                 dsems.at[lax.rem(j, 3)])

  pltpu.sync_copy(ei_hbm.at[:, pl.ds(base, CHUNK)], sidx_v.at[0])
  pltpu.sync_copy(w_hbm.at[pl.ds(base, CHUNK)], w_v.at[pl.ds(0, CHUNK)])
  gather(0).start()
  idx_copy(1).start()
  w_copy(1).start()
  idx_copy(1).wait()
  w_copy(1).wait()
  gather(1).start()
  idx_copy(2).start()
  w_copy(2).start()

  def chunk_body(j, carry):
    b = lax.rem(j, 2)

    @pl.when(j + 2 < NCHUNK)
    def _():
      idx_copy(j + 2).wait()
      w_copy(j + 2).wait()
      gather(j + 2).start()

    gather(j).wait()

    r3 = lax.rem(j, 3)
    @plsc.parallel_loop(0, CHUNK, unroll=4)
    def _(e):
      wv = plsc.load_gather(w_v, [jnp.full((16,), r3 * CHUNK + e, jnp.int32)])
      for k in range(D_IN // 32):
        x = msgs16_v[r3, e, pl.ds(k * 32, 32)]
        lo, hi = plsc.unpack(x, format=plsc.PackFormat.INTERLEAVED)
        msgsf_v[b, e, pl.ds(k * 32, 16)] = lo * wv
        msgsf_v[b, e, pl.ds(k * 32 + 16, 16)] = hi * wv

    @pl.when(j >= 1)
    def _():
      scat_h(j - 1).wait()
      scat_d(j - 1).wait()

    @pl.when(j + 3 < NCHUNK)
    def _():
      idx_copy(j + 3).start()
      w_copy(j + 3).start()

    scat_h(j).start(add=True)
    scat_d(j).start(add=True)
    return carry

  lax.fori_loop(0, NCHUNK, chunk_body, 0)

  scat_h(NCHUNK - 1).wait()
  scat_d(NCHUNK - 1).wait()

  plsc.subcore_barrier()

  @pl.when(s < NS - 1)
  def _():
    pltpu.sync_copy(aggh_sh.at[pl.ds(s * ROWS_FULL, ROWS_FULL)],
                    outh_hbm.at[c, pl.ds(s * ROWS_FULL, ROWS_FULL)])
    pltpu.sync_copy(aggd_sh.at[pl.ds(s * ROWS_FULL, ROWS_FULL)],
                    outd_hbm.at[c, pl.ds(s * ROWS_FULL, ROWS_FULL)])

  @pl.when(s == NS - 1)
  def _():
    pltpu.sync_copy(aggh_sh.at[pl.ds(15 * ROWS_FULL, ROWS_TAIL)],
                    outh_hbm.at[c, pl.ds(15 * ROWS_FULL, ROWS_TAIL)])
    pltpu.sync_copy(aggd_sh.at[pl.ds(15 * ROWS_FULL, ROWS_TAIL)],
                    outd_hbm.at[c, pl.ds(15 * ROWS_FULL, ROWS_TAIL)])


@functools.cache
def _edge_kernel():
  return pl.kernel(
      _edge_body,
      out_type=[
          jax.ShapeDtypeStruct((NC, N, D_IN), jnp.float32),
          jax.ShapeDtypeStruct((NC, N, D_DEG), jnp.float32),
      ],
      mesh=plsc.VectorSubcoreMesh(core_axis_name="c", subcore_axis_name="s",
                                  num_cores=NC, num_subcores=NS),
      compiler_params=pltpu.CompilerParams(needs_layout_passes=False,
                                           use_tc_tiling_on_sc=False),
      scratch_types=[
          pltpu.VMEM((4, 2, CHUNK), jnp.int32),
          pltpu.VMEM((3 * CHUNK,), jnp.float32),
          pltpu.VMEM((3, CHUNK, D_IN), jnp.bfloat16),
          pltpu.VMEM((2, CHUNK, D_IN), jnp.float32),
          pltpu.VMEM((CHUNK, D_DEG), jnp.float32),
          pltpu.VMEM((CHUNK, D_DEG), jnp.float32),
          pltpu.VMEM_SHARED((N, D_IN), jnp.float32),
          pltpu.VMEM_SHARED((N, D_DEG), jnp.float32),
          pltpu.SemaphoreType.DMA((2,)),
          pltpu.SemaphoreType.DMA((2,)),
          pltpu.SemaphoreType.DMA((3,)),
          pltpu.SemaphoreType.DMA((3,)),
      ],
  )


def _dense_body(feat_ref, h2_any, deg2_any, wself_ref, wneigh_ref, bsage_ref,
                gamma_ref, beta_ref, wfc_ref, bfc_ref, out_ref,
                h2_ref, deg2_ref, sem):
  pltpu.async_copy(h2_any, h2_ref, sem).wait()
  pltpu.async_copy(deg2_any, deg2_ref, sem).wait()
  h = h2_ref[0] + h2_ref[1]
  deg = deg2_ref[0, :, 0:1] + deg2_ref[1, :, 0:1]
  inv = 1.0 / jnp.maximum(deg, 1.0)
  h_neigh = jnp.dot(h * inv, wneigh_ref[...].T,
                    preferred_element_type=jnp.float32)
  rst = jnp.dot(feat_ref[...], wself_ref[...].T,
                preferred_element_type=jnp.float32)
  rst = rst + h_neigh + bsage_ref[...][None, :]
  rst = jnp.maximum(rst, 0.0)
  mean = jnp.mean(rst, axis=0, keepdims=True)
  var = jnp.mean((rst - mean) * (rst - mean), axis=0, keepdims=True)
  rst = (rst - mean) * jax.lax.rsqrt(var + 1e-5)
  rst = rst * gamma_ref[...][None, :] + beta_ref[...][None, :]
  out_ref[...] = jnp.dot(rst, wfc_ref[...].T,
                         preferred_element_type=jnp.float32) + bfc_ref[...][None, :]


_UNPACK_PERM = [32 * (q // 32) + 2 * (q % 32) if q % 32 < 16
                else 32 * (q // 32) + 2 * (q % 32 - 16) + 1
                for q in range(D_IN)]


def kernel(features, edge_weight, W_self, W_neigh, b_sage, bn_gamma, bn_beta,
           W_fc, b_fc, edge_index):
  feat16 = features.astype(jnp.bfloat16)

  h2, deg2 = _edge_kernel()(edge_index, edge_weight, feat16)

  wneigh_p = W_neigh[:, jnp.array(_UNPACK_PERM, jnp.int32)]

  vspec = pl.BlockSpec(memory_space=pltpu.VMEM)
  out = pl.pallas_call(
      _dense_body,
      out_shape=jax.ShapeDtypeStruct((N, D_OUT), jnp.float32),
      in_specs=[vspec,
                pl.BlockSpec(memory_space=pl.ANY),
                pl.BlockSpec(memory_space=pl.ANY),
                vspec, vspec, vspec, vspec, vspec, vspec, vspec],
      scratch_shapes=[
          pltpu.VMEM((NC, N, D_IN), jnp.float32),
          pltpu.VMEM((NC, N, D_DEG), jnp.float32),
          pltpu.SemaphoreType.DMA,
      ],
  )(features, h2, deg2, W_self, wneigh_p, b_sage, bn_gamma, bn_beta, W_fc, b_fc)
  return out
